# baseline (speedup 1.0000x reference)
import functools

import jax
import jax.numpy as jnp
import numpy as np
from jax import lax
from jax.experimental import pallas as pl
from jax.experimental.pallas import tpu as pltpu
from jax.experimental.pallas import tpu_sc as plsc

N_BINS = 1024
EMB_DIM = 64
PAD_DIM = 65
LO = np.float32(-np.pi)
SPAN = np.float32(np.pi - (-np.pi))

NC = 2
NS = 16
NW = NC * NS
LANES = 16

N_A = 16384
A_W = N_A // NW
A_CHK = 128
QB = 2
TAB_LEN = ((N_BINS + 1) * PAD_DIM + 127) // 128 * 128


def _body(theta_hbm, mask_hbm, table_hbm, out_hbm,
          tab_v, th_v, mk_v, idx_v, rows0_v, rows1_v, sem0, sem1):
    wid = lax.axis_index("c") * NS + lax.axis_index("s")
    pltpu.sync_copy(table_hbm, tab_v)
    lane = lax.iota(jnp.int32, LANES)

    def a_chunk(ar, carry):
        a0 = wid * A_W + ar * A_CHK
        r0 = a0 // 2
        pltpu.sync_copy(theta_hbm.at[pl.ds(r0, A_CHK // 2)], th_v)
        pltpu.sync_copy(mask_hbm.at[pl.ds(r0, A_CHK // 2)], mk_v)

        @plsc.parallel_loop(0, A_CHK * EMB_DIM // LANES, unroll=2)
        def bucketize(i):
            t = th_v[i // 8, pl.ds((i % 8) * LANES, LANES)]
            scaled = (t - LO) / SPAN * np.float32(N_BINS)
            bidx = scaled.astype(jnp.int32)
            bidx = jnp.minimum(jnp.maximum(bidx, 0), N_BINS - 1)
            m = mk_v[i // 8, pl.ds((i % 8) * LANES, LANES)]
            idx16 = jnp.where(m != 0, N_BINS, bidx)
            idx_v[pl.ds((i // 4) * PAD_DIM + (i % 4) * LANES, LANES)] = idx16

        def q_pair(t, carry2):
            for parity, (buf, sem) in enumerate(((rows0_v, sem0), (rows1_v, sem1))):
                q0 = 4 * t + 2 * parity
                dst = out_hbm.at[pl.ds(q0, QB), :, pl.ds(a0, A_CHK)]

                @pl.when(t > 0)
                def _drain():
                    pltpu.make_async_copy(buf, dst, sem).wait()

                @plsc.parallel_loop(0, A_CHK // LANES, unroll=4)
                def fill(a_sub):
                    lane65 = (a_sub * LANES + lane) * PAD_DIM
                    for q_l in range(QB):
                        a16 = plsc.load_gather(idx_v, [lane65 + (q0 + q_l)])
                        base16 = a16 * PAD_DIM
                        for c in range(EMB_DIM):
                            vals = plsc.load_gather(tab_v, [base16 + c])
                            buf[q_l, c, pl.ds(a_sub * LANES, LANES)] = vals

                pltpu.async_copy(buf, dst, sem)
            return carry2

        lax.fori_loop(0, EMB_DIM // (2 * QB), q_pair, 0)
        last = out_hbm.at[pl.ds(EMB_DIM - 2 * QB, QB), :, pl.ds(a0, A_CHK)]
        pltpu.make_async_copy(rows0_v, last, sem0).wait()
        pltpu.make_async_copy(rows1_v, last, sem1).wait()
        return carry

    lax.fori_loop(0, A_W // A_CHK, a_chunk, 0)


@functools.partial(jax.jit, static_argnames=())
def kernel(thetas, dist_0_mask, emb_table):
    theta_2d = thetas.reshape(N_A * EMB_DIM // 128, 128)
    mask_2d = dist_0_mask.reshape(N_A * EMB_DIM // 128, 128).astype(jnp.int32)
    tab_pad = jnp.pad(emb_table, ((0, 0), (0, PAD_DIM - EMB_DIM))).reshape(-1)
    tab_pad = jnp.pad(tab_pad, (0, TAB_LEN - tab_pad.shape[0]))
    mesh = plsc.VectorSubcoreMesh(core_axis_name="c", subcore_axis_name="s")
    run = pl.kernel(
        _body,
        out_type=jax.ShapeDtypeStruct((EMB_DIM, EMB_DIM, N_A), jnp.float32),
        mesh=mesh,
        scratch_types=[
            pltpu.VMEM((TAB_LEN,), jnp.float32),
            pltpu.VMEM((A_CHK // 2, 128), jnp.float32),
            pltpu.VMEM((A_CHK // 2, 128), jnp.int32),
            pltpu.VMEM((A_CHK * PAD_DIM,), jnp.int32),
            pltpu.VMEM((QB, EMB_DIM, A_CHK), jnp.float32),
            pltpu.VMEM((QB, EMB_DIM, A_CHK), jnp.float32),
            pltpu.SemaphoreType.DMA,
            pltpu.SemaphoreType.DMA,
        ],
        compiler_params=pltpu.CompilerParams(
            use_tc_tiling_on_sc=False, needs_layout_passes=False),
    )
    out = run(theta_2d, mask_2d, tab_pad)
    return jnp.transpose(out, (2, 0, 1))


# speedup vs baseline: 1.0452x; 1.0452x over previous
import functools

import jax
import jax.numpy as jnp
import numpy as np
from jax import lax
from jax.experimental import pallas as pl
from jax.experimental.pallas import tpu as pltpu
from jax.experimental.pallas import tpu_sc as plsc

N_BINS = 1024
EMB_DIM = 64
PAD_DIM = 65
LO = np.float32(-np.pi)
SPAN = np.float32(np.pi - (-np.pi))

NC = 2
NS = 16
NW = NC * NS
LANES = 16

N_A = 16384
A_W = N_A // NW
A_CHK = 128
QB = 2
TAB_LEN = ((N_BINS + 1) * PAD_DIM + 127) // 128 * 128


def _body(theta_hbm, mask_hbm, table_hbm, out_hbm,
          tab_v, th_v, mk_v, idx_v, rows0_v, rows1_v, sem0, sem1):
    wid = lax.axis_index("c") * NS + lax.axis_index("s")
    pltpu.sync_copy(table_hbm, tab_v)
    lane = lax.iota(jnp.int32, LANES)

    def a_chunk(ar, carry):
        a0 = wid * A_W + ar * A_CHK
        r0 = a0 // 2
        pltpu.sync_copy(theta_hbm.at[pl.ds(r0, A_CHK // 2)], th_v)
        pltpu.sync_copy(mask_hbm.at[pl.ds(r0, A_CHK // 2)], mk_v)

        @plsc.parallel_loop(0, A_CHK * EMB_DIM // LANES, unroll=2)
        def bucketize(i):
            t = th_v[i // 8, pl.ds((i % 8) * LANES, LANES)]
            scaled = (t - LO) / SPAN * np.float32(N_BINS)
            bidx = scaled.astype(jnp.int32)
            bidx = jnp.minimum(jnp.maximum(bidx, 0), N_BINS - 1)
            m = mk_v[i // 8, pl.ds((i % 8) * LANES, LANES)]
            idx16 = jnp.where(m != 0, N_BINS, bidx)
            idx_v[pl.ds((i // 4) * PAD_DIM + (i % 4) * LANES, LANES)] = idx16

        def q_pair(t, carry2):
            for parity, (buf, sem) in enumerate(((rows0_v, sem0), (rows1_v, sem1))):
                q0 = 4 * t + 2 * parity
                dst = out_hbm.at[pl.ds(q0, QB), :, pl.ds(a0, A_CHK)]

                @pl.when(t > 0)
                def _drain():
                    pltpu.make_async_copy(buf, dst, sem).wait()

                @plsc.parallel_loop(0, A_CHK // LANES, unroll=2)
                def fill(a_sub):
                    lane65 = (a_sub * LANES + lane) * PAD_DIM
                    for q_l in range(QB):
                        a16 = plsc.load_gather(idx_v, [lane65 + (q0 + q_l)])
                        base16 = a16 * PAD_DIM
                        for c in range(EMB_DIM):
                            vals = plsc.load_gather(tab_v, [base16 + c])
                            buf[q_l, c, pl.ds(a_sub * LANES, LANES)] = vals

                pltpu.async_copy(buf, dst, sem)
            return carry2

        lax.fori_loop(0, EMB_DIM // (2 * QB), q_pair, 0)
        last = out_hbm.at[pl.ds(EMB_DIM - 2 * QB, QB), :, pl.ds(a0, A_CHK)]
        pltpu.make_async_copy(rows0_v, last, sem0).wait()
        pltpu.make_async_copy(rows1_v, last, sem1).wait()
        return carry

    lax.fori_loop(0, A_W // A_CHK, a_chunk, 0)


@functools.partial(jax.jit, static_argnames=())
def kernel(thetas, dist_0_mask, emb_table):
    theta_2d = thetas.reshape(N_A * EMB_DIM // 128, 128)
    mask_2d = dist_0_mask.reshape(N_A * EMB_DIM // 128, 128).astype(jnp.int32)
    tab_pad = jnp.pad(emb_table, ((0, 0), (0, PAD_DIM - EMB_DIM))).reshape(-1)
    tab_pad = jnp.pad(tab_pad, (0, TAB_LEN - tab_pad.shape[0]))
    mesh = plsc.VectorSubcoreMesh(core_axis_name="c", subcore_axis_name="s")
    run = pl.kernel(
        _body,
        out_type=jax.ShapeDtypeStruct((EMB_DIM, EMB_DIM, N_A), jnp.float32),
        mesh=mesh,
        scratch_types=[
            pltpu.VMEM((TAB_LEN,), jnp.float32),
            pltpu.VMEM((A_CHK // 2, 128), jnp.float32),
            pltpu.VMEM((A_CHK // 2, 128), jnp.int32),
            pltpu.VMEM((A_CHK * PAD_DIM,), jnp.int32),
            pltpu.VMEM((QB, EMB_DIM, A_CHK), jnp.float32),
            pltpu.VMEM((QB, EMB_DIM, A_CHK), jnp.float32),
            pltpu.SemaphoreType.DMA,
            pltpu.SemaphoreType.DMA,
        ],
        compiler_params=pltpu.CompilerParams(
            use_tc_tiling_on_sc=False, needs_layout_passes=False),
    )
    out = run(theta_2d, mask_2d, tab_pad)
    return jnp.transpose(out, (2, 0, 1))
